# Initial kernel scaffold; baseline (speedup 1.0000x reference)
import functools

import jax
import jax.numpy as jnp
from jax import lax
from jax.experimental import pallas as pl
from jax.experimental.pallas import tpu as pltpu
from jax.experimental.pallas import tpu_sc as plsc

_N = 10000
_E = 160000
_D_IN = 128
_HID = 400
_D_OUT = 4

_NC = 2
_NS = 16
_NW = _NC * _NS
_L = 128
_EPAD = 163840
_ROWS_W = _EPAD // (_NW * _L)
_NACC = 10240
_SLICE = _NACC // _NS

_R = 1000
_HIDP = 512


def _sc_mesh():
    return plsc.VectorSubcoreMesh(core_axis_name="c", subcore_axis_name="s")


def _sc_agg1(x, src2d, dst2d, zrow, zcnt, ones_h):
    @functools.partial(
        pl.kernel,
        out_type=(
            jax.ShapeDtypeStruct((_NC, _NACC, _D_IN), jnp.float32),
            jax.ShapeDtypeStruct((_NC, _NACC, 16), jnp.float32),
        ),
        mesh=_sc_mesh(),
        scratch_types=[
            pltpu.VMEM((_ROWS_W, _L), jnp.int32),
            pltpu.VMEM((_ROWS_W, _L), jnp.int32),
            pltpu.VMEM((_L, _D_IN), jnp.float32),
            pltpu.VMEM((_L, 16), jnp.float32),
            pltpu.VMEM_SHARED((_NACC, _D_IN), jnp.float32),
            pltpu.VMEM_SHARED((_NACC, 16), jnp.float32),
            pltpu.SemaphoreType.DMA,
        ],
    )
    def k(x_h, src_h, dst_h, zrow_h, zcnt_h, ones_hh,
          acc_out, cnt_out, idxs, idxd, rows, ones_v, acc, cacc, sem):
        c = lax.axis_index("c")
        s = lax.axis_index("s")
        w = s * _NC + c
        pltpu.sync_copy(zrow_h, acc.at[pl.ds(s * _SLICE, _SLICE)])
        pltpu.sync_copy(zcnt_h, cacc.at[pl.ds(s * _SLICE, _SLICE)])
        pltpu.sync_copy(src_h.at[pl.ds(w * _ROWS_W, _ROWS_W)], idxs)
        pltpu.sync_copy(dst_h.at[pl.ds(w * _ROWS_W, _ROWS_W)], idxd)
        pltpu.sync_copy(ones_hh, ones_v)
        plsc.subcore_barrier()

        def body(kk, carry):
            pltpu.async_copy(x_h.at[idxs.at[kk]], rows, sem).wait()
            pltpu.sync_copy(rows, acc.at[idxd.at[kk]], add=True)
            pltpu.sync_copy(ones_v, cacc.at[idxd.at[kk]], add=True)
            return carry

        lax.fori_loop(0, _ROWS_W, body, 0)
        plsc.subcore_barrier()
        pltpu.sync_copy(acc.at[pl.ds(s * _SLICE, _SLICE)],
                        acc_out.at[c, pl.ds(s * _SLICE, _SLICE)])
        pltpu.sync_copy(cacc.at[pl.ds(s * _SLICE, _SLICE)],
                        cnt_out.at[c, pl.ds(s * _SLICE, _SLICE)])

    return k(x, src2d, dst2d, zrow, zcnt, ones_h)


def _sc_agg2(z, src2d, dst2d, zcnt):
    @functools.partial(
        pl.kernel,
        out_type=jax.ShapeDtypeStruct((_NC, _NACC, 16), jnp.float32),
        mesh=_sc_mesh(),
        scratch_types=[
            pltpu.VMEM((_ROWS_W, _L), jnp.int32),
            pltpu.VMEM((_ROWS_W, _L), jnp.int32),
            pltpu.VMEM((_L, 16), jnp.float32),
            pltpu.VMEM_SHARED((_NACC, 16), jnp.float32),
            pltpu.SemaphoreType.DMA,
        ],
    )
    def k(z_h, src_h, dst_h, zcnt_h, acc_out, idxs, idxd, rows, acc, sem):
        c = lax.axis_index("c")
        s = lax.axis_index("s")
        w = s * _NC + c
        pltpu.sync_copy(zcnt_h, acc.at[pl.ds(s * _SLICE, _SLICE)])
        pltpu.sync_copy(src_h.at[pl.ds(w * _ROWS_W, _ROWS_W)], idxs)
        pltpu.sync_copy(dst_h.at[pl.ds(w * _ROWS_W, _ROWS_W)], idxd)
        plsc.subcore_barrier()

        def body(kk, carry):
            pltpu.async_copy(z_h.at[idxs.at[kk]], rows, sem).wait()
            pltpu.sync_copy(rows, acc.at[idxd.at[kk]], add=True)
            return carry

        lax.fori_loop(0, _ROWS_W, body, 0)
        plsc.subcore_barrier()
        pltpu.sync_copy(acc.at[pl.ds(s * _SLICE, _SLICE)],
                        acc_out.at[c, pl.ds(s * _SLICE, _SLICE)])

    return k(z, src2d, dst2d, zcnt)


def _fuse1_body(pacc_ref, pcnt_ref, x_ref, w1_ref, b1_ref, w2_ref,
                z_ref, deg_ref):
    a = pacc_ref[0] + pacc_ref[1] + x_ref[...]
    deg = pcnt_ref[0, :, 0:1] + pcnt_ref[1, :, 0:1] + 1.0
    mean = a / deg
    h = jnp.maximum(
        jnp.dot(mean, w1_ref[...], preferred_element_type=jnp.float32)
        + b1_ref[...], 0.0)
    z = jnp.dot(h, w2_ref[...], preferred_element_type=jnp.float32)
    z_ref[...] = z
    deg_ref[...] = jnp.broadcast_to(deg, deg_ref.shape)


def _fuse1(pacc, pcnt, x, w1p, b1p, w2p):
    return pl.pallas_call(
        _fuse1_body,
        grid=(_N // _R,),
        in_specs=[
            pl.BlockSpec((2, _R, _D_IN), lambda i: (0, i, 0)),
            pl.BlockSpec((2, _R, 16), lambda i: (0, i, 0)),
            pl.BlockSpec((_R, _D_IN), lambda i: (i, 0)),
            pl.BlockSpec((_D_IN, _HIDP), lambda i: (0, 0)),
            pl.BlockSpec((1, _HIDP), lambda i: (0, 0)),
            pl.BlockSpec((_HIDP, 16), lambda i: (0, 0)),
        ],
        out_specs=[
            pl.BlockSpec((_R, 16), lambda i: (i, 0)),
            pl.BlockSpec((_R, 16), lambda i: (i, 0)),
        ],
        out_shape=[
            jax.ShapeDtypeStruct((_N, 16), jnp.float32),
            jax.ShapeDtypeStruct((_N, 16), jnp.float32),
        ],
    )(pacc, pcnt, x, w1p, b1p, w2p)


def _fuse2_body(q_ref, z_ref, deg_ref, b2_ref, o_ref):
    ssum = q_ref[0] + q_ref[1] + z_ref[...]
    o = jnp.maximum(ssum / deg_ref[...] + b2_ref[...], 0.0)
    o_ref[...] = o[:, 0:4]


def _fuse2(q, z, deg, b2p):
    return pl.pallas_call(
        _fuse2_body,
        grid=(_N // _R,),
        in_specs=[
            pl.BlockSpec((2, _R, 16), lambda i: (0, i, 0)),
            pl.BlockSpec((_R, 16), lambda i: (i, 0)),
            pl.BlockSpec((_R, 16), lambda i: (i, 0)),
            pl.BlockSpec((1, 16), lambda i: (0, 0)),
        ],
        out_specs=pl.BlockSpec((_R, _D_OUT), lambda i: (i, 0)),
        out_shape=jax.ShapeDtypeStruct((_N, _D_OUT), jnp.float32),
    )(q, z, deg, b2p)


@jax.jit
def kernel(x, edge_index, W1, U1, c1, b1, W2, U2, c2, b2):
    src = edge_index[0]
    dst = edge_index[1]
    dst_eff = jnp.where(src != dst, dst, _N)
    pad = _EPAD - _E
    src_p = jnp.concatenate([src, jnp.zeros((pad,), jnp.int32)])
    dst_p = jnp.concatenate([dst_eff, jnp.full((pad,), _N, jnp.int32)])
    src2d = src_p.reshape(_EPAD // _L, _L)
    dst2d = dst_p.reshape(_EPAD // _L, _L)

    zrow = jnp.zeros((_SLICE, _D_IN), jnp.float32)
    zcnt = jnp.zeros((_SLICE, 16), jnp.float32)
    ones_h = jnp.ones((_L, 16), jnp.float32)

    w1p = jnp.pad(W1, ((0, 0), (0, _HIDP - _HID)))
    b1p = jnp.pad(b1, (0, _HIDP - _HID)).reshape(1, _HIDP)
    w2p = jnp.pad(W2, ((0, _HIDP - _HID), (0, 16 - _D_OUT)))
    b2p = jnp.pad(b2, (0, 16 - _D_OUT)).reshape(1, 16)

    pacc, pcnt = _sc_agg1(x, src2d, dst2d, zrow, zcnt, ones_h)
    z, deg = _fuse1(pacc, pcnt, x, w1p, b1p, w2p)
    q = _sc_agg2(z, src2d, dst2d, zcnt)
    return _fuse2(q, z, deg, b2p)


# speedup vs baseline: 13.8911x; 13.8911x over previous
import functools

import jax
import jax.numpy as jnp
from jax import lax
from jax.experimental import pallas as pl
from jax.experimental.pallas import tpu as pltpu
from jax.experimental.pallas import tpu_sc as plsc

_N = 10000
_E = 160000
_D_IN = 128
_HID = 400
_D_OUT = 4

_NC = 2
_NS = 16
_NW = _NC * _NS
_L = 128
_EPAD = 163840
_EROWS = _EPAD // _L
_ROWS_CORE = _EROWS // _NS
_ROWS_W = _EROWS // _NW
_NACC = 10240
_SLICE = _NACC // _NS
_DH = 80

_R = 1000
_HIDP = 512


def _sc_mesh():
    return plsc.VectorSubcoreMesh(core_axis_name="c", subcore_axis_name="s")


def _sc_agg1(xcat, src3d, dst2d, zrow):
    @functools.partial(
        pl.kernel,
        out_type=jax.ShapeDtypeStruct((_NC, _NACC, _DH), jnp.float32),
        mesh=_sc_mesh(),
        compiler_params=pltpu.CompilerParams(use_tc_tiling_on_sc=False),
        scratch_types=[
            pltpu.VMEM((_ROWS_CORE, _L), jnp.int32),
            pltpu.VMEM((_ROWS_CORE, _L), jnp.int32),
            pltpu.VMEM((_L, _DH), jnp.float32),
            pltpu.VMEM_SHARED((_NACC, _DH), jnp.float32),
            pltpu.SemaphoreType.DMA,
        ],
    )
    def k(x_h, src_h, dst_h, zrow_h, acc_out, idxs, idxd, rows, acc, sem):
        c = lax.axis_index("c")
        s = lax.axis_index("s")
        pltpu.sync_copy(zrow_h, acc.at[pl.ds(s * _SLICE, _SLICE)])
        pltpu.sync_copy(src_h.at[c, pl.ds(s * _ROWS_CORE, _ROWS_CORE)], idxs)
        pltpu.sync_copy(dst_h.at[pl.ds(s * _ROWS_CORE, _ROWS_CORE)], idxd)
        plsc.subcore_barrier()

        def body(kk, carry):
            pltpu.async_copy(x_h.at[idxs.at[kk]], rows, sem).wait()
            pltpu.sync_copy(rows, acc.at[idxd.at[kk]], add=True)
            return carry

        lax.fori_loop(0, _ROWS_CORE, body, 0)
        plsc.subcore_barrier()
        pltpu.sync_copy(acc.at[pl.ds(s * _SLICE, _SLICE)],
                        acc_out.at[c, pl.ds(s * _SLICE, _SLICE)])

    return k(xcat, src3d, dst2d, zrow)


def _sc_agg2(z, src2d, dst2d, zcnt):
    @functools.partial(
        pl.kernel,
        out_type=jax.ShapeDtypeStruct((_NC, _NACC, 16), jnp.float32),
        mesh=_sc_mesh(),
        compiler_params=pltpu.CompilerParams(use_tc_tiling_on_sc=False),
        scratch_types=[
            pltpu.VMEM((_ROWS_W, _L), jnp.int32),
            pltpu.VMEM((_ROWS_W, _L), jnp.int32),
            pltpu.VMEM((_L, 16), jnp.float32),
            pltpu.VMEM_SHARED((_NACC, 16), jnp.float32),
            pltpu.SemaphoreType.DMA,
        ],
    )
    def k(z_h, src_h, dst_h, zcnt_h, acc_out, idxs, idxd, rows, acc, sem):
        c = lax.axis_index("c")
        s = lax.axis_index("s")
        w = s * _NC + c
        pltpu.sync_copy(zcnt_h, acc.at[pl.ds(s * _SLICE, _SLICE)])
        pltpu.sync_copy(src_h.at[pl.ds(w * _ROWS_W, _ROWS_W)], idxs)
        pltpu.sync_copy(dst_h.at[pl.ds(w * _ROWS_W, _ROWS_W)], idxd)
        plsc.subcore_barrier()

        def body(kk, carry):
            pltpu.async_copy(z_h.at[idxs.at[kk]], rows, sem).wait()
            pltpu.sync_copy(rows, acc.at[idxd.at[kk]], add=True)
            return carry

        lax.fori_loop(0, _ROWS_W, body, 0)
        plsc.subcore_barrier()
        pltpu.sync_copy(acc.at[pl.ds(s * _SLICE, _SLICE)],
                        acc_out.at[c, pl.ds(s * _SLICE, _SLICE)])

    return k(z, src2d, dst2d, zcnt)


def _fuse1_body(pacc_ref, x_ref, w1_ref, b1_ref, w2_ref, z_ref, deg_ref):
    agg = jnp.concatenate([pacc_ref[0], pacc_ref[1, :, 0:48]], axis=1)
    a = agg + x_ref[...]
    deg = pacc_ref[1, :, 48:49] + 1.0
    mean = a / deg
    h = jnp.maximum(
        jnp.dot(mean, w1_ref[...], preferred_element_type=jnp.float32)
        + b1_ref[...], 0.0)
    z = jnp.dot(h, w2_ref[...], preferred_element_type=jnp.float32)
    z_ref[...] = z
    deg_ref[...] = jnp.broadcast_to(deg, deg_ref.shape)


def _fuse1(pacc, x, w1p, b1p, w2p):
    return pl.pallas_call(
        _fuse1_body,
        grid=(_N // _R,),
        in_specs=[
            pl.BlockSpec((2, _R, _DH), lambda i: (0, i, 0)),
            pl.BlockSpec((_R, _D_IN), lambda i: (i, 0)),
            pl.BlockSpec((_D_IN, _HIDP), lambda i: (0, 0)),
            pl.BlockSpec((1, _HIDP), lambda i: (0, 0)),
            pl.BlockSpec((_HIDP, 16), lambda i: (0, 0)),
        ],
        out_specs=[
            pl.BlockSpec((_R, 16), lambda i: (i, 0)),
            pl.BlockSpec((_R, 16), lambda i: (i, 0)),
        ],
        out_shape=[
            jax.ShapeDtypeStruct((_N, 16), jnp.float32),
            jax.ShapeDtypeStruct((_N, 16), jnp.float32),
        ],
    )(pacc, x, w1p, b1p, w2p)


def _fuse2_body(q_ref, z_ref, deg_ref, b2_ref, o_ref):
    ssum = q_ref[0] + q_ref[1] + z_ref[...]
    o = jnp.maximum(ssum / deg_ref[...] + b2_ref[...], 0.0)
    o_ref[...] = o[:, 0:4]


def _fuse2(q, z, deg, b2p):
    return pl.pallas_call(
        _fuse2_body,
        grid=(_N // _R,),
        in_specs=[
            pl.BlockSpec((2, _R, 16), lambda i: (0, i, 0)),
            pl.BlockSpec((_R, 16), lambda i: (i, 0)),
            pl.BlockSpec((_R, 16), lambda i: (i, 0)),
            pl.BlockSpec((1, 16), lambda i: (0, 0)),
        ],
        out_specs=pl.BlockSpec((_R, _D_OUT), lambda i: (i, 0)),
        out_shape=jax.ShapeDtypeStruct((_N, _D_OUT), jnp.float32),
    )(q, z, deg, b2p)


@jax.jit
def kernel(x, edge_index, W1, U1, c1, b1, W2, U2, c2, b2):
    src = edge_index[0]
    dst = edge_index[1]
    dst_eff = jnp.where(src != dst, dst, _N)
    pad = _EPAD - _E
    src_p = jnp.concatenate([src, jnp.zeros((pad,), jnp.int32)])
    dst_p = jnp.concatenate([dst_eff, jnp.full((pad,), _N, jnp.int32)])
    src2d = src_p.reshape(_EROWS, _L)
    dst2d = dst_p.reshape(_EROWS, _L)
    src3d = jnp.stack([src2d, src2d + _N])

    xaug = jnp.concatenate(
        [x, jnp.ones((_N, 1), jnp.float32), jnp.zeros((_N, 31), jnp.float32)],
        axis=1)
    xcat = jnp.concatenate([xaug[:, :_DH], xaug[:, _DH:]], axis=0)

    zrow = jnp.zeros((_SLICE, _DH), jnp.float32)
    zcnt = jnp.zeros((_SLICE, 16), jnp.float32)

    w1p = jnp.pad(W1, ((0, 0), (0, _HIDP - _HID)))
    b1p = jnp.pad(b1, (0, _HIDP - _HID)).reshape(1, _HIDP)
    w2p = jnp.pad(W2, ((0, _HIDP - _HID), (0, 16 - _D_OUT)))
    b2p = jnp.pad(b2, (0, 16 - _D_OUT)).reshape(1, 16)

    pacc = _sc_agg1(xcat, src3d, dst2d, zrow)
    z, deg = _fuse1(pacc, x, w1p, b1p, w2p)
    q = _sc_agg2(z, src2d, dst2d, zcnt)
    return _fuse2(q, z, deg, b2p)


# speedup vs baseline: 16.8851x; 1.2155x over previous
import functools

import jax
import jax.numpy as jnp
from jax import lax
from jax.experimental import pallas as pl
from jax.experimental.pallas import tpu as pltpu
from jax.experimental.pallas import tpu_sc as plsc

_N = 10000
_E = 160000
_D_IN = 128
_HID = 400
_D_OUT = 4

_NC = 2
_NS = 16
_NW = _NC * _NS
_L = 128
_EPAD = 163840
_EROWS = _EPAD // _L
_ROWS_CORE = _EROWS // _NS
_ROWS_W = _EROWS // _NW
_NACC = 10240
_SLICE = _NACC // _NS
_DH = 80

_R = 1000
_HIDP = 512


def _sc_mesh():
    return plsc.VectorSubcoreMesh(core_axis_name="c", subcore_axis_name="s")


def _sc_agg1(xcat, src3d, dst2d, zrow):
    @functools.partial(
        pl.kernel,
        out_type=jax.ShapeDtypeStruct((_NC, _NACC, _DH), jnp.float32),
        mesh=_sc_mesh(),
        compiler_params=pltpu.CompilerParams(use_tc_tiling_on_sc=False),
        scratch_types=[
            pltpu.VMEM((_ROWS_CORE, _L), jnp.int32),
            pltpu.VMEM((_ROWS_CORE, _L), jnp.int32),
            pltpu.VMEM((_L, _DH), jnp.float32),
            pltpu.VMEM((_L, _DH), jnp.float32),
            pltpu.VMEM_SHARED((_NACC, _DH), jnp.float32),
            pltpu.SemaphoreType.DMA,
            pltpu.SemaphoreType.DMA,
        ],
    )
    def k(x_h, src_h, dst_h, zrow_h, acc_out,
          idxs, idxd, rows0, rows1, acc, sem0, sem1):
        c = lax.axis_index("c")
        s = lax.axis_index("s")
        pltpu.sync_copy(zrow_h, acc.at[pl.ds(s * _SLICE, _SLICE)])
        pltpu.sync_copy(src_h.at[c, pl.ds(s * _ROWS_CORE, _ROWS_CORE)], idxs)
        pltpu.sync_copy(dst_h.at[pl.ds(s * _ROWS_CORE, _ROWS_CORE)], idxd)
        plsc.subcore_barrier()

        pltpu.async_copy(x_h.at[idxs.at[0]], rows0, sem0)

        def body(j, carry):
            pltpu.async_copy(x_h.at[idxs.at[2 * j + 1]], rows1, sem1)
            pltpu.make_async_copy(x_h.at[idxs.at[2 * j]], rows0, sem0).wait()
            pltpu.sync_copy(rows0, acc.at[idxd.at[2 * j]], add=True)

            @pl.when(j < _ROWS_CORE // 2 - 1)
            def _():
                pltpu.async_copy(x_h.at[idxs.at[2 * j + 2]], rows0, sem0)

            pltpu.make_async_copy(x_h.at[idxs.at[2 * j + 1]], rows1, sem1).wait()
            pltpu.sync_copy(rows1, acc.at[idxd.at[2 * j + 1]], add=True)
            return carry

        lax.fori_loop(0, _ROWS_CORE // 2, body, 0)
        plsc.subcore_barrier()
        pltpu.sync_copy(acc.at[pl.ds(s * _SLICE, _SLICE)],
                        acc_out.at[c, pl.ds(s * _SLICE, _SLICE)])

    return k(xcat, src3d, dst2d, zrow)


def _sc_agg2(z, src2d, dst2d, zcnt):
    @functools.partial(
        pl.kernel,
        out_type=jax.ShapeDtypeStruct((_NC, _NACC, 16), jnp.float32),
        mesh=_sc_mesh(),
        compiler_params=pltpu.CompilerParams(use_tc_tiling_on_sc=False),
        scratch_types=[
            pltpu.VMEM((_ROWS_W, _L), jnp.int32),
            pltpu.VMEM((_ROWS_W, _L), jnp.int32),
            pltpu.VMEM((_L, 16), jnp.float32),
            pltpu.VMEM((_L, 16), jnp.float32),
            pltpu.VMEM_SHARED((_NACC, 16), jnp.float32),
            pltpu.SemaphoreType.DMA,
            pltpu.SemaphoreType.DMA,
        ],
    )
    def k(z_h, src_h, dst_h, zcnt_h, acc_out,
          idxs, idxd, rows0, rows1, acc, sem0, sem1):
        c = lax.axis_index("c")
        s = lax.axis_index("s")
        w = s * _NC + c
        pltpu.sync_copy(zcnt_h, acc.at[pl.ds(s * _SLICE, _SLICE)])
        pltpu.sync_copy(src_h.at[pl.ds(w * _ROWS_W, _ROWS_W)], idxs)
        pltpu.sync_copy(dst_h.at[pl.ds(w * _ROWS_W, _ROWS_W)], idxd)
        plsc.subcore_barrier()

        pltpu.async_copy(z_h.at[idxs.at[0]], rows0, sem0)

        def body(j, carry):
            pltpu.async_copy(z_h.at[idxs.at[2 * j + 1]], rows1, sem1)
            pltpu.make_async_copy(z_h.at[idxs.at[2 * j]], rows0, sem0).wait()
            pltpu.sync_copy(rows0, acc.at[idxd.at[2 * j]], add=True)

            @pl.when(j < _ROWS_W // 2 - 1)
            def _():
                pltpu.async_copy(z_h.at[idxs.at[2 * j + 2]], rows0, sem0)

            pltpu.make_async_copy(z_h.at[idxs.at[2 * j + 1]], rows1, sem1).wait()
            pltpu.sync_copy(rows1, acc.at[idxd.at[2 * j + 1]], add=True)
            return carry

        lax.fori_loop(0, _ROWS_W // 2, body, 0)
        plsc.subcore_barrier()
        pltpu.sync_copy(acc.at[pl.ds(s * _SLICE, _SLICE)],
                        acc_out.at[c, pl.ds(s * _SLICE, _SLICE)])

    return k(z, src2d, dst2d, zcnt)


def _fuse1_body(pacc_ref, x_ref, w1_ref, b1_ref, w2_ref, z_ref, deg_ref):
    agg = jnp.concatenate([pacc_ref[0], pacc_ref[1, :, 0:48]], axis=1)
    a = agg + x_ref[...]
    deg = pacc_ref[1, :, 48:49] + 1.0
    mean = a / deg
    h = jnp.maximum(
        jnp.dot(mean, w1_ref[...], preferred_element_type=jnp.float32)
        + b1_ref[...], 0.0)
    z = jnp.dot(h, w2_ref[...], preferred_element_type=jnp.float32)
    z_ref[...] = z
    deg_ref[...] = jnp.broadcast_to(deg, deg_ref.shape)


def _fuse1(pacc, x, w1p, b1p, w2p):
    return pl.pallas_call(
        _fuse1_body,
        grid=(_N // _R,),
        in_specs=[
            pl.BlockSpec((2, _R, _DH), lambda i: (0, i, 0)),
            pl.BlockSpec((_R, _D_IN), lambda i: (i, 0)),
            pl.BlockSpec((_D_IN, _HIDP), lambda i: (0, 0)),
            pl.BlockSpec((1, _HIDP), lambda i: (0, 0)),
            pl.BlockSpec((_HIDP, 16), lambda i: (0, 0)),
        ],
        out_specs=[
            pl.BlockSpec((_R, 16), lambda i: (i, 0)),
            pl.BlockSpec((_R, 16), lambda i: (i, 0)),
        ],
        out_shape=[
            jax.ShapeDtypeStruct((_N, 16), jnp.float32),
            jax.ShapeDtypeStruct((_N, 16), jnp.float32),
        ],
    )(pacc, x, w1p, b1p, w2p)


def _fuse2_body(q_ref, z_ref, deg_ref, b2_ref, o_ref):
    ssum = q_ref[0] + q_ref[1] + z_ref[...]
    o = jnp.maximum(ssum / deg_ref[...] + b2_ref[...], 0.0)
    o_ref[...] = o[:, 0:4]


def _fuse2(q, z, deg, b2p):
    return pl.pallas_call(
        _fuse2_body,
        grid=(_N // _R,),
        in_specs=[
            pl.BlockSpec((2, _R, 16), lambda i: (0, i, 0)),
            pl.BlockSpec((_R, 16), lambda i: (i, 0)),
            pl.BlockSpec((_R, 16), lambda i: (i, 0)),
            pl.BlockSpec((1, 16), lambda i: (0, 0)),
        ],
        out_specs=pl.BlockSpec((_R, _D_OUT), lambda i: (i, 0)),
        out_shape=jax.ShapeDtypeStruct((_N, _D_OUT), jnp.float32),
    )(q, z, deg, b2p)


@jax.jit
def kernel(x, edge_index, W1, U1, c1, b1, W2, U2, c2, b2):
    src = edge_index[0]
    dst = edge_index[1]
    dst_eff = jnp.where(src != dst, dst, _N)
    pad = _EPAD - _E
    src_p = jnp.concatenate([src, jnp.zeros((pad,), jnp.int32)])
    dst_p = jnp.concatenate([dst_eff, jnp.full((pad,), _N, jnp.int32)])
    src2d = src_p.reshape(_EROWS, _L)
    dst2d = dst_p.reshape(_EROWS, _L)
    src3d = jnp.stack([src2d, src2d + _N])

    xaug = jnp.concatenate(
        [x, jnp.ones((_N, 1), jnp.float32), jnp.zeros((_N, 31), jnp.float32)],
        axis=1)
    xcat = jnp.concatenate([xaug[:, :_DH], xaug[:, _DH:]], axis=0)

    zrow = jnp.zeros((_SLICE, _DH), jnp.float32)
    zcnt = jnp.zeros((_SLICE, 16), jnp.float32)

    w1p = jnp.pad(W1, ((0, 0), (0, _HIDP - _HID)))
    b1p = jnp.pad(b1, (0, _HIDP - _HID)).reshape(1, _HIDP)
    w2p = jnp.pad(W2, ((0, _HIDP - _HID), (0, 16 - _D_OUT)))
    b2p = jnp.pad(b2, (0, 16 - _D_OUT)).reshape(1, 16)

    pacc = _sc_agg1(xcat, src3d, dst2d, zrow)
    z, deg = _fuse1(pacc, x, w1p, b1p, w2p)
    q = _sc_agg2(z, src2d, dst2d, zcnt)
    return _fuse2(q, z, deg, b2p)


# speedup vs baseline: 16.9657x; 1.0048x over previous
import functools

import jax
import jax.numpy as jnp
from jax import lax
from jax.experimental import pallas as pl
from jax.experimental.pallas import tpu as pltpu
from jax.experimental.pallas import tpu_sc as plsc

_N = 10000
_E = 160000
_D_IN = 128
_HID = 400
_D_OUT = 4

_NC = 2
_NS = 16
_NW = _NC * _NS
_L = 128
_EPAD = 163840
_EROWS = _EPAD // _L
_ROWS_CORE = _EROWS // _NS
_ROWS_W = _EROWS // _NW
_NACC = 10240
_SLICE = _NACC // _NS
_DH = 80

_R = 1000
_HIDP = 512


def _sc_mesh():
    return plsc.VectorSubcoreMesh(core_axis_name="c", subcore_axis_name="s")


def _agg_pipeline(t_h, idxs, idxd, acc, rs, semg, sems, nch):
    def gather(kk, b):
        pltpu.async_copy(t_h.at[idxs.at[kk]], rs[b], semg[b])

    def wait_scatter(kk, b):
        pltpu.make_async_copy(rs[b], acc.at[idxd.at[kk]], sems[b]).wait()

    gather(0, 0)
    gather(1, 1)
    nj = nch // 4

    def body(j, carry):
        for i in range(4):
            kk = 4 * j + i
            bn = (i + 2) % 4
            pltpu.make_async_copy(t_h.at[idxs.at[kk]], rs[i], semg[i]).wait()
            pltpu.async_copy(rs[i], acc.at[idxd.at[kk]], sems[i], add=True)
            if i < 2:
                @pl.when(j > 0)
                def _():
                    wait_scatter(kk - 2, bn)
                gather(kk + 2, bn)
            else:
                wait_scatter(kk - 2, bn)

                @pl.when(j < nj - 1)
                def _():
                    gather(kk + 2, bn)
        return carry

    lax.fori_loop(0, nj, body, 0)
    wait_scatter(nch - 2, 2)
    wait_scatter(nch - 1, 3)


def _sc_agg1(xcat, src3d, dst2d, zrow):
    @functools.partial(
        pl.kernel,
        out_type=jax.ShapeDtypeStruct((_NC, _NACC, _DH), jnp.float32),
        mesh=_sc_mesh(),
        compiler_params=pltpu.CompilerParams(use_tc_tiling_on_sc=False),
        scratch_types=[
            pltpu.VMEM((_ROWS_CORE, _L), jnp.int32),
            pltpu.VMEM((_ROWS_CORE, _L), jnp.int32),
            pltpu.VMEM((_L, _DH), jnp.float32),
            pltpu.VMEM((_L, _DH), jnp.float32),
            pltpu.VMEM((_L, _DH), jnp.float32),
            pltpu.VMEM((_L, _DH), jnp.float32),
            pltpu.VMEM_SHARED((_NACC, _DH), jnp.float32),
            pltpu.SemaphoreType.DMA,
            pltpu.SemaphoreType.DMA,
            pltpu.SemaphoreType.DMA,
            pltpu.SemaphoreType.DMA,
            pltpu.SemaphoreType.DMA,
            pltpu.SemaphoreType.DMA,
            pltpu.SemaphoreType.DMA,
            pltpu.SemaphoreType.DMA,
        ],
    )
    def k(x_h, src_h, dst_h, zrow_h, acc_out, idxs, idxd,
          r0, r1, r2, r3, acc, g0, g1, g2, g3, s0, s1, s2, s3):
        c = lax.axis_index("c")
        s = lax.axis_index("s")
        pltpu.sync_copy(zrow_h, acc.at[pl.ds(s * _SLICE, _SLICE)])
        pltpu.sync_copy(src_h.at[c, pl.ds(s * _ROWS_CORE, _ROWS_CORE)], idxs)
        pltpu.sync_copy(dst_h.at[pl.ds(s * _ROWS_CORE, _ROWS_CORE)], idxd)
        plsc.subcore_barrier()
        _agg_pipeline(x_h, idxs, idxd, acc, (r0, r1, r2, r3),
                      (g0, g1, g2, g3), (s0, s1, s2, s3), _ROWS_CORE)
        plsc.subcore_barrier()
        pltpu.sync_copy(acc.at[pl.ds(s * _SLICE, _SLICE)],
                        acc_out.at[c, pl.ds(s * _SLICE, _SLICE)])

    return k(xcat, src3d, dst2d, zrow)


def _sc_agg2(z, src2d, dst2d, zcnt):
    @functools.partial(
        pl.kernel,
        out_type=jax.ShapeDtypeStruct((_NC, _NACC, 16), jnp.float32),
        mesh=_sc_mesh(),
        compiler_params=pltpu.CompilerParams(use_tc_tiling_on_sc=False),
        scratch_types=[
            pltpu.VMEM((_ROWS_W, _L), jnp.int32),
            pltpu.VMEM((_ROWS_W, _L), jnp.int32),
            pltpu.VMEM((_L, 16), jnp.float32),
            pltpu.VMEM((_L, 16), jnp.float32),
            pltpu.VMEM((_L, 16), jnp.float32),
            pltpu.VMEM((_L, 16), jnp.float32),
            pltpu.VMEM_SHARED((_NACC, 16), jnp.float32),
            pltpu.SemaphoreType.DMA,
            pltpu.SemaphoreType.DMA,
            pltpu.SemaphoreType.DMA,
            pltpu.SemaphoreType.DMA,
            pltpu.SemaphoreType.DMA,
            pltpu.SemaphoreType.DMA,
            pltpu.SemaphoreType.DMA,
            pltpu.SemaphoreType.DMA,
        ],
    )
    def k(z_h, src_h, dst_h, zcnt_h, acc_out, idxs, idxd,
          r0, r1, r2, r3, acc, g0, g1, g2, g3, s0, s1, s2, s3):
        c = lax.axis_index("c")
        s = lax.axis_index("s")
        w = s * _NC + c
        pltpu.sync_copy(zcnt_h, acc.at[pl.ds(s * _SLICE, _SLICE)])
        pltpu.sync_copy(src_h.at[pl.ds(w * _ROWS_W, _ROWS_W)], idxs)
        pltpu.sync_copy(dst_h.at[pl.ds(w * _ROWS_W, _ROWS_W)], idxd)
        plsc.subcore_barrier()
        _agg_pipeline(z_h, idxs, idxd, acc, (r0, r1, r2, r3),
                      (g0, g1, g2, g3), (s0, s1, s2, s3), _ROWS_W)
        plsc.subcore_barrier()
        pltpu.sync_copy(acc.at[pl.ds(s * _SLICE, _SLICE)],
                        acc_out.at[c, pl.ds(s * _SLICE, _SLICE)])

    return k(z, src2d, dst2d, zcnt)


def _fuse1_body(pacc_ref, x_ref, w1_ref, b1_ref, w2_ref, z_ref, deg_ref):
    agg = jnp.concatenate([pacc_ref[0], pacc_ref[1, :, 0:48]], axis=1)
    a = agg + x_ref[...]
    deg = pacc_ref[1, :, 48:49] + 1.0
    mean = a / deg
    h = jnp.maximum(
        jnp.dot(mean, w1_ref[...], preferred_element_type=jnp.float32)
        + b1_ref[...], 0.0)
    z = jnp.dot(h, w2_ref[...], preferred_element_type=jnp.float32)
    z_ref[...] = z
    deg_ref[...] = jnp.broadcast_to(deg, deg_ref.shape)


def _fuse1(pacc, x, w1p, b1p, w2p):
    return pl.pallas_call(
        _fuse1_body,
        grid=(_N // _R,),
        in_specs=[
            pl.BlockSpec((2, _R, _DH), lambda i: (0, i, 0)),
            pl.BlockSpec((_R, _D_IN), lambda i: (i, 0)),
            pl.BlockSpec((_D_IN, _HIDP), lambda i: (0, 0)),
            pl.BlockSpec((1, _HIDP), lambda i: (0, 0)),
            pl.BlockSpec((_HIDP, 16), lambda i: (0, 0)),
        ],
        out_specs=[
            pl.BlockSpec((_R, 16), lambda i: (i, 0)),
            pl.BlockSpec((_R, 16), lambda i: (i, 0)),
        ],
        out_shape=[
            jax.ShapeDtypeStruct((_N, 16), jnp.float32),
            jax.ShapeDtypeStruct((_N, 16), jnp.float32),
        ],
    )(pacc, x, w1p, b1p, w2p)


def _fuse2_body(q_ref, z_ref, deg_ref, b2_ref, o_ref):
    ssum = q_ref[0] + q_ref[1] + z_ref[...]
    o = jnp.maximum(ssum / deg_ref[...] + b2_ref[...], 0.0)
    o_ref[...] = o[:, 0:4]


def _fuse2(q, z, deg, b2p):
    return pl.pallas_call(
        _fuse2_body,
        grid=(_N // _R,),
        in_specs=[
            pl.BlockSpec((2, _R, 16), lambda i: (0, i, 0)),
            pl.BlockSpec((_R, 16), lambda i: (i, 0)),
            pl.BlockSpec((_R, 16), lambda i: (i, 0)),
            pl.BlockSpec((1, 16), lambda i: (0, 0)),
        ],
        out_specs=pl.BlockSpec((_R, _D_OUT), lambda i: (i, 0)),
        out_shape=jax.ShapeDtypeStruct((_N, _D_OUT), jnp.float32),
    )(q, z, deg, b2p)


@jax.jit
def kernel(x, edge_index, W1, U1, c1, b1, W2, U2, c2, b2):
    src = edge_index[0]
    dst = edge_index[1]
    dst_eff = jnp.where(src != dst, dst, _N)
    pad = _EPAD - _E
    src_p = jnp.concatenate([src, jnp.zeros((pad,), jnp.int32)])
    dst_p = jnp.concatenate([dst_eff, jnp.full((pad,), _N, jnp.int32)])
    src2d = src_p.reshape(_EROWS, _L)
    dst2d = dst_p.reshape(_EROWS, _L)
    src3d = jnp.stack([src2d, src2d + _N])

    xaug = jnp.concatenate(
        [x, jnp.ones((_N, 1), jnp.float32), jnp.zeros((_N, 31), jnp.float32)],
        axis=1)
    xcat = jnp.concatenate([xaug[:, :_DH], xaug[:, _DH:]], axis=0)

    zrow = jnp.zeros((_SLICE, _DH), jnp.float32)
    zcnt = jnp.zeros((_SLICE, 16), jnp.float32)

    w1p = jnp.pad(W1, ((0, 0), (0, _HIDP - _HID)))
    b1p = jnp.pad(b1, (0, _HIDP - _HID)).reshape(1, _HIDP)
    w2p = jnp.pad(W2, ((0, _HIDP - _HID), (0, 16 - _D_OUT)))
    b2p = jnp.pad(b2, (0, 16 - _D_OUT)).reshape(1, 16)

    pacc = _sc_agg1(xcat, src3d, dst2d, zrow)
    z, deg = _fuse1(pacc, x, w1p, b1p, w2p)
    q = _sc_agg2(z, src2d, dst2d, zcnt)
    return _fuse2(q, z, deg, b2p)
